# baseline (speedup 1.0000x reference)
import functools

import jax
import jax.numpy as jnp
from jax.experimental import pallas as pl
from jax.experimental.pallas import tpu as pltpu

_VMEM_LIMIT_BYTES = 64 * 1024 * 1024


def _pool_kernel(x_ref, o_ref, *, inv_area):
    s = (jnp.sum(x_ref[...], axis=0) * inv_area).astype(o_ref.dtype)
    o_ref[...] = s.reshape(o_ref.shape)


def kernel(x):
    n, c, h, w = x.shape
    hw = h * w
    inv_area = 1.0 / float(hw)

    xt = jnp.transpose(x, (2, 3, 0, 1)).reshape(hw, n, c)

    bn = n
    for cand in (32, 16, 8, 4, 2, 1):
        if n % cand == 0:
            bn = cand
            break
    num_tiles = n // bn

    if c % 128 == 0:
        out_shape = jax.ShapeDtypeStruct((n, c // 128, 128), x.dtype)
        out_spec = pl.BlockSpec((bn, c // 128, 128), lambda i: (i, 0, 0))
    else:
        out_shape = jax.ShapeDtypeStruct((n, c), x.dtype)
        out_spec = pl.BlockSpec((bn, c), lambda i: (i, 0))

    out = pl.pallas_call(
        functools.partial(_pool_kernel, inv_area=inv_area),
        out_shape=out_shape,
        grid=(num_tiles,),
        in_specs=[pl.BlockSpec((hw, bn, c), lambda i: (0, i, 0))],
        out_specs=out_spec,
        compiler_params=pltpu.CompilerParams(
            dimension_semantics=("parallel",),
            vmem_limit_bytes=_VMEM_LIMIT_BYTES,
        ),
    )(xt)

    return out.reshape(n, c, 1, 1)


# speedup vs baseline: 1.0441x; 1.0441x over previous
import functools

import jax
import jax.numpy as jnp
from jax.experimental import pallas as pl
from jax.experimental.pallas import tpu as pltpu

_VMEM_LIMIT_BYTES = 64 * 1024 * 1024


def _pool_kernel(x_ref, o_ref, *, inv_area):
    s = (jnp.sum(x_ref[...], axis=0) * inv_area).astype(o_ref.dtype)
    o_ref[...] = s.reshape(o_ref.shape)


def kernel(x):
    n, c, h, w = x.shape
    hw = h * w
    inv_area = 1.0 / float(hw)

    xt = jnp.transpose(x, (2, 3, 0, 1)).reshape(hw, n, c)

    bn = n
    for cand in (16, 8, 4, 2, 1):
        if n % cand == 0:
            bn = cand
            break
    num_tiles = n // bn

    if c % 128 == 0:
        out_shape = jax.ShapeDtypeStruct((n, c // 128, 128), x.dtype)
        out_spec = pl.BlockSpec((bn, c // 128, 128), lambda i: (i, 0, 0))
    else:
        out_shape = jax.ShapeDtypeStruct((n, c), x.dtype)
        out_spec = pl.BlockSpec((bn, c), lambda i: (i, 0))

    out = pl.pallas_call(
        functools.partial(_pool_kernel, inv_area=inv_area),
        out_shape=out_shape,
        grid=(num_tiles,),
        in_specs=[pl.BlockSpec((hw, bn, c), lambda i: (0, i, 0))],
        out_specs=out_spec,
        compiler_params=pltpu.CompilerParams(
            dimension_semantics=("parallel",),
            vmem_limit_bytes=_VMEM_LIMIT_BYTES,
        ),
    )(xt)

    return out.reshape(n, c, 1, 1)
